# baseline (speedup 1.0000x reference)
import functools

import jax
import jax.numpy as jnp
from jax import lax
from jax.experimental import pallas as pl
from jax.experimental.pallas import tpu as pltpu
from jax.experimental.pallas import tpu_sc as plsc

N = 10000
D = 128
E = 320000
HEADS = 4
ATT = 128
CLS = 128
NUM_CLASSES = 7

TILES = 32
CHUNK = 128
NCHUNK = 79
EPT = NCHUNK * CHUNK
EPAD = TILES * EPT
ACC_ROWS = 10240
ROW_BLK = 5000
GRID = N // ROW_BLK

_HI = jax.lax.Precision.HIGHEST


def _segsum_body(h_hbm, src_hbm, dst_hbm, out_hbm,
                 src_t, dst_t, rows, acc, gsem):
    cid = lax.axis_index("c")
    sid = lax.axis_index("s")
    wid = cid * 16 + sid

    def _zrow(i, carry):
        for c8 in range(D // 16):
            rows[i, pl.ds(c8 * 16, 16)] = jnp.zeros((16,), jnp.float32)
        return carry
    lax.fori_loop(0, CHUNK, _zrow, 0)

    def _zchunk(k, carry):
        pltpu.sync_copy(rows, acc.at[pl.ds((sid * 5 + k) * CHUNK, CHUNK)])
        return carry
    lax.fori_loop(0, ACC_ROWS // CHUNK // 16, _zchunk, 0)

    pltpu.sync_copy(src_hbm.at[wid], src_t)
    pltpu.sync_copy(dst_hbm.at[wid], dst_t)

    plsc.subcore_barrier()

    def _step(j, carry):
        pltpu.async_copy(h_hbm.at[src_t.at[j]], rows, gsem).wait()
        pltpu.sync_copy(rows, acc.at[dst_t.at[j]], add=True)
        return carry
    lax.fori_loop(0, NCHUNK, _step, 0)

    plsc.subcore_barrier()

    def _ochunk(k, carry):
        r0 = (sid * 5 + k) * CHUNK
        pltpu.sync_copy(acc.at[pl.ds(r0, CHUNK)], rows)
        pltpu.sync_copy(rows, out_hbm.at[cid, pl.ds(r0, CHUNK)])
        return carry
    lax.fori_loop(0, ACC_ROWS // CHUNK // 16, _ochunk, 0)


_segsum = functools.partial(
    pl.kernel,
    out_type=jax.ShapeDtypeStruct((2, ACC_ROWS, D), jnp.float32),
    mesh=plsc.VectorSubcoreMesh(core_axis_name="c", subcore_axis_name="s"),
    scratch_types=[
        pltpu.VMEM((NCHUNK, CHUNK), jnp.int32),
        pltpu.VMEM((NCHUNK, CHUNK), jnp.int32),
        pltpu.VMEM((CHUNK, D), jnp.float32),
        pltpu.VMEM_SHARED((ACC_ROWS, D), jnp.float32),
        pltpu.SemaphoreType.DMA,
    ],
)(_segsum_body)


def _ln(v, w, b, eps=1e-5):
    mu = jnp.mean(v, axis=-1, keepdims=True)
    var = jnp.mean((v - mu) ** 2, axis=-1, keepdims=True)
    return (v - mu) * jax.lax.rsqrt(var + eps) * w + b


def _gin_body(sc_ref, h_ref, a0_ref, a1_ref,
              w1_ref, b1_ref, g1_ref, e1_ref,
              w2_ref, b2_ref, g2_ref, e2_ref, o_ref):
    h = h_ref[...]
    z = sc_ref[0, 0] * h + a0_ref[0] + a1_ref[0]
    z = jnp.dot(z, w1_ref[...], precision=_HI) + b1_ref[...]
    z = jnp.maximum(_ln(z, g1_ref[...], e1_ref[...]), 0.0)
    z = jnp.dot(z, w2_ref[...], precision=_HI) + b2_ref[...]
    z = jnp.maximum(_ln(z, g2_ref[...], e2_ref[...]), 0.0)
    o_ref[...] = z + h


def _gin_dense(scale, h, agg, p, ln_w, ln_b):
    full = lambda s: pl.BlockSpec(s, lambda i: (0, 0))
    return pl.pallas_call(
        _gin_body,
        grid=(GRID,),
        in_specs=[
            full((1, 1)),
            pl.BlockSpec((ROW_BLK, D), lambda i: (i, 0)),
            pl.BlockSpec((1, ROW_BLK, D), lambda i: (0, i, 0)),
            pl.BlockSpec((1, ROW_BLK, D), lambda i: (1, i, 0)),
            full((D, D)), full((1, D)), full((1, D)), full((1, D)),
            full((D, D)), full((1, D)), full((1, D)), full((1, D)),
        ],
        out_specs=pl.BlockSpec((ROW_BLK, D), lambda i: (i, 0)),
        out_shape=jax.ShapeDtypeStruct((N, D), jnp.float32),
    )(scale, h, agg, agg,
      p['W1'], p['b1'].reshape(1, D), p['ln1_w'].reshape(1, D),
      p['ln1_b'].reshape(1, D),
      p['W2'], p['b2'].reshape(1, D), ln_w.reshape(1, D), ln_b.reshape(1, D))


def _gin_score_body(sc_ref, h_ref, a0_ref, a1_ref,
                    w1_ref, b1_ref, g1_ref, e1_ref,
                    w2_ref, b2_ref, g2_ref, e2_ref,
                    aw1_ref, ab1_ref, aw2_ref, ab2_ref,
                    o_ref, s_ref):
    h = h_ref[...]
    z = sc_ref[0, 0] * h + a0_ref[0] + a1_ref[0]
    z = jnp.dot(z, w1_ref[...], precision=_HI) + b1_ref[...]
    z = jnp.maximum(_ln(z, g1_ref[...], e1_ref[...]), 0.0)
    z = jnp.dot(z, w2_ref[...], precision=_HI) + b2_ref[...]
    z = jnp.maximum(_ln(z, g2_ref[...], e2_ref[...]), 0.0)
    h2 = z + h
    o_ref[...] = h2
    t = jnp.tanh(jnp.dot(h2, aw1_ref[...], precision=_HI) + ab1_ref[...])
    tr = t.reshape(ROW_BLK, HEADS, ATT)
    s_ref[...] = jnp.sum(tr * aw2_ref[...][None], axis=-1) + ab2_ref[...]


def _gin_dense_scores(scale, h, agg, p, ln_w, ln_b, w1c, b1c, w2c, b2c):
    full = lambda s: pl.BlockSpec(s, lambda i: (0, 0))
    return pl.pallas_call(
        _gin_score_body,
        grid=(GRID,),
        in_specs=[
            full((1, 1)),
            pl.BlockSpec((ROW_BLK, D), lambda i: (i, 0)),
            pl.BlockSpec((1, ROW_BLK, D), lambda i: (0, i, 0)),
            pl.BlockSpec((1, ROW_BLK, D), lambda i: (1, i, 0)),
            full((D, D)), full((1, D)), full((1, D)), full((1, D)),
            full((D, D)), full((1, D)), full((1, D)), full((1, D)),
            full((D, HEADS * ATT)), full((1, HEADS * ATT)),
            full((HEADS, ATT)), full((1, HEADS)),
        ],
        out_specs=[
            pl.BlockSpec((ROW_BLK, D), lambda i: (i, 0)),
            pl.BlockSpec((ROW_BLK, HEADS), lambda i: (i, 0)),
        ],
        out_shape=[
            jax.ShapeDtypeStruct((N, D), jnp.float32),
            jax.ShapeDtypeStruct((N, HEADS), jnp.float32),
        ],
    )(scale, h, agg, agg,
      p['W1'], p['b1'].reshape(1, D), p['ln1_w'].reshape(1, D),
      p['ln1_b'].reshape(1, D),
      p['W2'], p['b2'].reshape(1, D), ln_w.reshape(1, D), ln_b.reshape(1, D),
      w1c, b1c, w2c, b2c)


def _pool_body(s_ref, h_ref,
               w1_ref, b1_ref, g1_ref, e1_ref,
               w2_ref, b2_ref, g2_ref, e2_ref,
               w3_ref, b3_ref,
               att_ref, probs_ref, u_ref, ml_ref):
    i = pl.program_id(0)

    @pl.when(i == 0)
    def _init():
        s = s_ref[...]
        m = jnp.max(s, axis=0, keepdims=True)
        l = jnp.sum(jnp.exp(s - m), axis=0, keepdims=True)
        ml_ref[0:1, :] = m
        ml_ref[1:2, :] = l
        u_ref[...] = jnp.zeros_like(u_ref)

    m = ml_ref[0:1, :]
    l = ml_ref[1:2, :]
    e = jnp.exp(s_ref[pl.ds(i * ROW_BLK, ROW_BLK), :] - m)
    att_ref[...] = e / l
    u_ref[...] += lax.dot_general(e, h_ref[...], (((0,), (0,)), ((), ())),
                                  precision=_HI)

    @pl.when(i == pl.num_programs(0) - 1)
    def _final():
        recip = (1.0 / HEADS) / l
        z = jnp.dot(recip, u_ref[...], precision=_HI)
        t = jnp.dot(z, w1_ref[...], precision=_HI) + b1_ref[...]
        t = jnp.maximum(_ln(t, g1_ref[...], e1_ref[...]), 0.0)
        t = jnp.dot(t, w2_ref[...], precision=_HI) + b2_ref[...]
        t = jnp.maximum(_ln(t, g2_ref[...], e2_ref[...]), 0.0)
        lg = jnp.dot(t, w3_ref[...], precision=_HI) + b3_ref[...]
        mm = jnp.max(lg, axis=-1, keepdims=True)
        p = jnp.exp(lg - mm)
        probs_ref[...] = p / jnp.sum(p, axis=-1, keepdims=True)


def _pool_cls(s, h, c):
    full = lambda shape: pl.BlockSpec(shape, lambda i: (0, 0))
    return pl.pallas_call(
        _pool_body,
        grid=(GRID,),
        in_specs=[
            full((N, HEADS)),
            pl.BlockSpec((ROW_BLK, D), lambda i: (i, 0)),
            full((D, CLS)), full((1, CLS)), full((1, CLS)), full((1, CLS)),
            full((CLS, CLS // 2)), full((1, CLS // 2)),
            full((1, CLS // 2)), full((1, CLS // 2)),
            full((CLS // 2, NUM_CLASSES)), full((1, NUM_CLASSES)),
        ],
        out_specs=[
            pl.BlockSpec((ROW_BLK, HEADS), lambda i: (i, 0)),
            full((1, NUM_CLASSES)),
        ],
        out_shape=[
            jax.ShapeDtypeStruct((N, HEADS), jnp.float32),
            jax.ShapeDtypeStruct((1, NUM_CLASSES), jnp.float32),
        ],
        scratch_shapes=[
            pltpu.VMEM((HEADS, D), jnp.float32),
            pltpu.VMEM((2, HEADS), jnp.float32),
        ],
    )(s, h,
      c['W1'], c['b1'].reshape(1, CLS), c['ln1_w'].reshape(1, CLS),
      c['ln1_b'].reshape(1, CLS),
      c['W2'], c['b2'].reshape(1, CLS // 2), c['ln2_w'].reshape(1, CLS // 2),
      c['ln2_b'].reshape(1, CLS // 2),
      c['W3'], c['b3'].reshape(1, NUM_CLASSES))


def kernel(x, edge_index, params):
    src = edge_index[0]
    dst = edge_index[1]
    pad = EPAD - E
    trash = N + jnp.arange(pad, dtype=jnp.int32) % (ACC_ROWS - N)
    src_r = jnp.concatenate([src, jnp.zeros((pad,), jnp.int32)])
    src_r = src_r.reshape(TILES, NCHUNK, CHUNK)
    dst_r = jnp.concatenate([dst, trash]).reshape(TILES, NCHUNK, CHUNK)

    att_p = params['att']
    w1c = jnp.concatenate([hp['W1'] for hp in att_p], axis=1)
    b1c = jnp.concatenate([hp['b1'] for hp in att_p]).reshape(1, HEADS * ATT)
    w2c = jnp.stack([hp['W2'][:, 0] for hp in att_p], axis=0)
    b2c = jnp.stack([hp['b2'][0] for hp in att_p]).reshape(1, HEADS)

    p0 = params['gin'][0]
    agg = _segsum(x, src_r, dst_r)
    h = _gin_dense((1.0 + p0['eps']).reshape(1, 1), x, agg, p0,
                   params['ln_w'][0], params['ln_b'][0])

    p1 = params['gin'][1]
    agg = _segsum(h, src_r, dst_r)
    h, s = _gin_dense_scores((1.0 + p1['eps']).reshape(1, 1), h, agg, p1,
                             params['ln_w'][1], params['ln_b'][1],
                             w1c, b1c, w2c, b2c)

    att, probs = _pool_cls(s, h, params['cls'])
    return (probs.reshape(NUM_CLASSES), att)


# speedup vs baseline: 1.1032x; 1.1032x over previous
import functools

import jax
import jax.numpy as jnp
from jax import lax
from jax.experimental import pallas as pl
from jax.experimental.pallas import tpu as pltpu
from jax.experimental.pallas import tpu_sc as plsc

N = 10000
D = 128
E = 320000
HEADS = 4
ATT = 128
CLS = 128
NUM_CLASSES = 7

TILES = 32
CHUNK = 128
NCHUNK = 79
EPT = NCHUNK * CHUNK
EPAD = TILES * EPT
ACC_ROWS = 10240
ROW_BLK = 2000
GRID = N // ROW_BLK

_HI = jax.lax.Precision.HIGHEST


def _segsum_body(h_hbm, src_hbm, dst_hbm, out_hbm,
                 src_t, dst_t, rows, acc, gsem):
    cid = lax.axis_index("c")
    sid = lax.axis_index("s")
    wid = cid * 16 + sid

    def _zrow(i, carry):
        for c8 in range(D // 16):
            rows[i, pl.ds(c8 * 16, 16)] = jnp.zeros((16,), jnp.float32)
        return carry
    lax.fori_loop(0, CHUNK, _zrow, 0)

    def _zchunk(k, carry):
        pltpu.sync_copy(rows, acc.at[pl.ds((sid * 5 + k) * CHUNK, CHUNK)])
        return carry
    lax.fori_loop(0, ACC_ROWS // CHUNK // 16, _zchunk, 0)

    pltpu.sync_copy(src_hbm.at[wid], src_t)
    pltpu.sync_copy(dst_hbm.at[wid], dst_t)

    plsc.subcore_barrier()

    def _step(j, carry):
        pltpu.async_copy(h_hbm.at[src_t.at[j]], rows, gsem).wait()
        pltpu.sync_copy(rows, acc.at[dst_t.at[j]], add=True)
        return carry
    lax.fori_loop(0, NCHUNK, _step, 0)

    plsc.subcore_barrier()

    def _ochunk(k, carry):
        r0 = (sid * 5 + k) * CHUNK
        pltpu.sync_copy(acc.at[pl.ds(r0, CHUNK)], rows)
        pltpu.sync_copy(rows, out_hbm.at[cid, pl.ds(r0, CHUNK)])
        return carry
    lax.fori_loop(0, ACC_ROWS // CHUNK // 16, _ochunk, 0)


_segsum = functools.partial(
    pl.kernel,
    out_type=jax.ShapeDtypeStruct((2, ACC_ROWS, D), jnp.float32),
    mesh=plsc.VectorSubcoreMesh(core_axis_name="c", subcore_axis_name="s"),
    scratch_types=[
        pltpu.VMEM((NCHUNK, CHUNK), jnp.int32),
        pltpu.VMEM((NCHUNK, CHUNK), jnp.int32),
        pltpu.VMEM((CHUNK, D), jnp.float32),
        pltpu.VMEM_SHARED((ACC_ROWS, D), jnp.float32),
        pltpu.SemaphoreType.DMA,
    ],
)(_segsum_body)


def _ln(v, w, b, eps=1e-5):
    mu = jnp.mean(v, axis=-1, keepdims=True)
    var = jnp.mean((v - mu) ** 2, axis=-1, keepdims=True)
    return (v - mu) * jax.lax.rsqrt(var + eps) * w + b


def _gin_body(sc_ref, h_ref, a0_ref, a1_ref,
              w1_ref, b1_ref, g1_ref, e1_ref,
              w2_ref, b2_ref, g2_ref, e2_ref, o_ref):
    h = h_ref[...]
    z = sc_ref[0, 0] * h + a0_ref[0] + a1_ref[0]
    z = jnp.dot(z, w1_ref[...], precision=_HI) + b1_ref[...]
    z = jnp.maximum(_ln(z, g1_ref[...], e1_ref[...]), 0.0)
    z = jnp.dot(z, w2_ref[...], precision=_HI) + b2_ref[...]
    z = jnp.maximum(_ln(z, g2_ref[...], e2_ref[...]), 0.0)
    o_ref[...] = z + h


def _gin_dense(scale, h, agg, p, ln_w, ln_b):
    full = lambda s: pl.BlockSpec(s, lambda i: (0, 0))
    return pl.pallas_call(
        _gin_body,
        grid=(GRID,),
        in_specs=[
            full((1, 1)),
            pl.BlockSpec((ROW_BLK, D), lambda i: (i, 0)),
            pl.BlockSpec((1, ROW_BLK, D), lambda i: (0, i, 0)),
            pl.BlockSpec((1, ROW_BLK, D), lambda i: (1, i, 0)),
            full((D, D)), full((1, D)), full((1, D)), full((1, D)),
            full((D, D)), full((1, D)), full((1, D)), full((1, D)),
        ],
        out_specs=pl.BlockSpec((ROW_BLK, D), lambda i: (i, 0)),
        out_shape=jax.ShapeDtypeStruct((N, D), jnp.float32),
    )(scale, h, agg, agg,
      p['W1'], p['b1'].reshape(1, D), p['ln1_w'].reshape(1, D),
      p['ln1_b'].reshape(1, D),
      p['W2'], p['b2'].reshape(1, D), ln_w.reshape(1, D), ln_b.reshape(1, D))


def _gin_score_body(sc_ref, h_ref, a0_ref, a1_ref,
                    w1_ref, b1_ref, g1_ref, e1_ref,
                    w2_ref, b2_ref, g2_ref, e2_ref,
                    aw1_ref, ab1_ref, aw2_ref, ab2_ref,
                    o_ref, s_ref):
    h = h_ref[...]
    z = sc_ref[0, 0] * h + a0_ref[0] + a1_ref[0]
    z = jnp.dot(z, w1_ref[...], precision=_HI) + b1_ref[...]
    z = jnp.maximum(_ln(z, g1_ref[...], e1_ref[...]), 0.0)
    z = jnp.dot(z, w2_ref[...], precision=_HI) + b2_ref[...]
    z = jnp.maximum(_ln(z, g2_ref[...], e2_ref[...]), 0.0)
    h2 = z + h
    o_ref[...] = h2
    t = jnp.tanh(jnp.dot(h2, aw1_ref[...], precision=_HI) + ab1_ref[...])
    tr = t.reshape(ROW_BLK, HEADS, ATT)
    s_ref[...] = jnp.sum(tr * aw2_ref[...][None], axis=-1) + ab2_ref[...]


def _gin_dense_scores(scale, h, agg, p, ln_w, ln_b, w1c, b1c, w2c, b2c):
    full = lambda s: pl.BlockSpec(s, lambda i: (0, 0))
    return pl.pallas_call(
        _gin_score_body,
        grid=(GRID,),
        in_specs=[
            full((1, 1)),
            pl.BlockSpec((ROW_BLK, D), lambda i: (i, 0)),
            pl.BlockSpec((1, ROW_BLK, D), lambda i: (0, i, 0)),
            pl.BlockSpec((1, ROW_BLK, D), lambda i: (1, i, 0)),
            full((D, D)), full((1, D)), full((1, D)), full((1, D)),
            full((D, D)), full((1, D)), full((1, D)), full((1, D)),
            full((D, HEADS * ATT)), full((1, HEADS * ATT)),
            full((HEADS, ATT)), full((1, HEADS)),
        ],
        out_specs=[
            pl.BlockSpec((ROW_BLK, D), lambda i: (i, 0)),
            pl.BlockSpec((ROW_BLK, HEADS), lambda i: (i, 0)),
        ],
        out_shape=[
            jax.ShapeDtypeStruct((N, D), jnp.float32),
            jax.ShapeDtypeStruct((N, HEADS), jnp.float32),
        ],
    )(scale, h, agg, agg,
      p['W1'], p['b1'].reshape(1, D), p['ln1_w'].reshape(1, D),
      p['ln1_b'].reshape(1, D),
      p['W2'], p['b2'].reshape(1, D), ln_w.reshape(1, D), ln_b.reshape(1, D),
      w1c, b1c, w2c, b2c)


def _pool_body(s_ref, h_ref,
               w1_ref, b1_ref, g1_ref, e1_ref,
               w2_ref, b2_ref, g2_ref, e2_ref,
               w3_ref, b3_ref,
               att_ref, probs_ref, u_ref, ml_ref):
    i = pl.program_id(0)

    @pl.when(i == 0)
    def _init():
        s = s_ref[...]
        m = jnp.max(s, axis=0, keepdims=True)
        l = jnp.sum(jnp.exp(s - m), axis=0, keepdims=True)
        ml_ref[0:1, :] = m
        ml_ref[1:2, :] = l
        u_ref[...] = jnp.zeros_like(u_ref)

    m = ml_ref[0:1, :]
    l = ml_ref[1:2, :]
    e = jnp.exp(s_ref[pl.ds(i * ROW_BLK, ROW_BLK), :] - m)
    att_ref[...] = e / l
    u_ref[...] += lax.dot_general(e, h_ref[...], (((0,), (0,)), ((), ())),
                                  precision=_HI)

    @pl.when(i == pl.num_programs(0) - 1)
    def _final():
        recip = (1.0 / HEADS) / l
        z = jnp.dot(recip, u_ref[...], precision=_HI)
        t = jnp.dot(z, w1_ref[...], precision=_HI) + b1_ref[...]
        t = jnp.maximum(_ln(t, g1_ref[...], e1_ref[...]), 0.0)
        t = jnp.dot(t, w2_ref[...], precision=_HI) + b2_ref[...]
        t = jnp.maximum(_ln(t, g2_ref[...], e2_ref[...]), 0.0)
        lg = jnp.dot(t, w3_ref[...], precision=_HI) + b3_ref[...]
        mm = jnp.max(lg, axis=-1, keepdims=True)
        p = jnp.exp(lg - mm)
        probs_ref[...] = p / jnp.sum(p, axis=-1, keepdims=True)


def _pool_cls(s, h, c):
    full = lambda shape: pl.BlockSpec(shape, lambda i: (0, 0))
    return pl.pallas_call(
        _pool_body,
        grid=(GRID,),
        in_specs=[
            full((N, HEADS)),
            pl.BlockSpec((ROW_BLK, D), lambda i: (i, 0)),
            full((D, CLS)), full((1, CLS)), full((1, CLS)), full((1, CLS)),
            full((CLS, CLS // 2)), full((1, CLS // 2)),
            full((1, CLS // 2)), full((1, CLS // 2)),
            full((CLS // 2, NUM_CLASSES)), full((1, NUM_CLASSES)),
        ],
        out_specs=[
            pl.BlockSpec((ROW_BLK, HEADS), lambda i: (i, 0)),
            full((1, NUM_CLASSES)),
        ],
        out_shape=[
            jax.ShapeDtypeStruct((N, HEADS), jnp.float32),
            jax.ShapeDtypeStruct((1, NUM_CLASSES), jnp.float32),
        ],
        scratch_shapes=[
            pltpu.VMEM((HEADS, D), jnp.float32),
            pltpu.VMEM((2, HEADS), jnp.float32),
        ],
    )(s, h,
      c['W1'], c['b1'].reshape(1, CLS), c['ln1_w'].reshape(1, CLS),
      c['ln1_b'].reshape(1, CLS),
      c['W2'], c['b2'].reshape(1, CLS // 2), c['ln2_w'].reshape(1, CLS // 2),
      c['ln2_b'].reshape(1, CLS // 2),
      c['W3'], c['b3'].reshape(1, NUM_CLASSES))


def kernel(x, edge_index, params):
    src = edge_index[0]
    dst = edge_index[1]
    pad = EPAD - E
    trash = N + jnp.arange(pad, dtype=jnp.int32) % (ACC_ROWS - N)
    src_r = jnp.concatenate([src, jnp.zeros((pad,), jnp.int32)])
    src_r = src_r.reshape(TILES, NCHUNK, CHUNK)
    dst_r = jnp.concatenate([dst, trash]).reshape(TILES, NCHUNK, CHUNK)

    att_p = params['att']
    w1c = jnp.concatenate([hp['W1'] for hp in att_p], axis=1)
    b1c = jnp.concatenate([hp['b1'] for hp in att_p]).reshape(1, HEADS * ATT)
    w2c = jnp.stack([hp['W2'][:, 0] for hp in att_p], axis=0)
    b2c = jnp.stack([hp['b2'][0] for hp in att_p]).reshape(1, HEADS)

    p0 = params['gin'][0]
    agg = _segsum(x, src_r, dst_r)
    h = _gin_dense((1.0 + p0['eps']).reshape(1, 1), x, agg, p0,
                   params['ln_w'][0], params['ln_b'][0])

    p1 = params['gin'][1]
    agg = _segsum(h, src_r, dst_r)
    h, s = _gin_dense_scores((1.0 + p1['eps']).reshape(1, 1), h, agg, p1,
                             params['ln_w'][1], params['ln_b'][1],
                             w1c, b1c, w2c, b2c)

    att, probs = _pool_cls(s, h, params['cls'])
    return (probs.reshape(NUM_CLASSES), att)
